# baseline (speedup 1.0000x reference)
import functools

import jax
import jax.numpy as jnp
from jax import lax
from jax.experimental import pallas as pl
from jax.experimental.pallas import tpu as pltpu
from jax.experimental.pallas import tpu_sc as plsc

NC = 2
NS = 16
NW = NC * NS
LANES = 16
CHUNK = 128


NBUF = 3


def _sc_agg(x, sd, wp, zrows):
    N, D = x.shape
    _, K, _, C = sd.shape
    assert D % LANES == 0 and K % NBUF == 0 and K >= NBUF
    STRIPE = 640
    LAST = N - STRIPE * (NS - 1)
    assert zrows.shape[0] == STRIPE and LAST > 0
    nfeat = D // LANES

    mesh = plsc.VectorSubcoreMesh(core_axis_name="c", subcore_axis_name="s")

    @functools.partial(
        pl.kernel,
        out_type=jax.ShapeDtypeStruct((NC, N, D), jnp.float32),
        mesh=mesh,
        scratch_types=(
            [pltpu.VMEM((2, C), jnp.int32)] * NBUF
            + [pltpu.VMEM((C,), jnp.float32)] * NBUF
            + [pltpu.VMEM((C, D), jnp.float32)] * NBUF
            + [pltpu.VMEM_SHARED((N, D), jnp.float32)]
            + [pltpu.SemaphoreType.DMA] * (2 * NBUF)
        ),
        compiler_params=pltpu.CompilerParams(needs_layout_passes=False),
    )
    def k(x_hbm, sd_hbm, w_hbm, z_hbm, out_hbm,
          e0, e1, e2, w0, w1, w2, r0, r1, r2, agg_sh,
          is0, is1, is2, gs0, gs1, gs2):
        ebuf = (e0, e1, e2)
        wbuf = (w0, w1, w2)
        rows = (r0, r1, r2)
        isem = (is0, is1, is2)
        gsem = (gs0, gs1, gs2)
        c = lax.axis_index("c")
        s = lax.axis_index("s")
        wid = s * NC + c
        base = pl.multiple_of(s * STRIPE, 8)

        def issue_idx(q, b):
            pltpu.async_copy(sd_hbm.at[wid, q], ebuf[b], isem[b])
            pltpu.async_copy(w_hbm.at[wid, q], wbuf[b], isem[b])

        def wait_idx(b):
            pltpu.make_async_copy(sd_hbm.at[wid, 0], ebuf[b], isem[b]).wait()
            pltpu.make_async_copy(w_hbm.at[wid, 0], wbuf[b], isem[b]).wait()

        def issue_gather(b):
            pltpu.async_copy(x_hbm.at[ebuf[b].at[0]], rows[b], gsem[b])

        def wait_gather(b):
            pltpu.make_async_copy(
                x_hbm.at[ebuf[b].at[0]], rows[b], gsem[b]).wait()

        for b in range(NBUF):
            issue_idx(b, b)

        @pl.when(s < NS - 1)
        def _():
            pltpu.sync_copy(z_hbm.at[pl.ds(0, STRIPE)],
                            agg_sh.at[pl.ds(base, STRIPE)])

        @pl.when(s == NS - 1)
        def _():
            pltpu.sync_copy(z_hbm.at[pl.ds(0, LAST)],
                            agg_sh.at[pl.ds(base, LAST)])
        plsc.subcore_barrier()

        wait_idx(0)
        issue_gather(0)

        def tri(t, _):
            for b in range(NBUF):
                q = t * NBUF + b
                b1 = (b + 1) % NBUF

                @pl.when(q + 1 < K)
                def _():
                    wait_idx(b1)
                    issue_gather(b1)

                wait_gather(b)

                def edge(e, _):
                    wspl = plsc.load_gather(
                        wbuf[b], [jnp.full((LANES,), e, jnp.int32)])
                    for f in range(nfeat):
                        sl = pl.ds(f * LANES, LANES)
                        rows[b][e, sl] = rows[b][e, sl] * wspl
                    return 0
                lax.fori_loop(0, C, edge, 0)

                pltpu.sync_copy(rows[b], agg_sh.at[ebuf[b].at[1]], add=True)

                @pl.when(q + NBUF < K)
                def _():
                    issue_idx(q + NBUF, b)
            return 0
        lax.fori_loop(0, K // NBUF, tri, 0)

        plsc.subcore_barrier()

        @pl.when(s < NS - 1)
        def _():
            pltpu.sync_copy(agg_sh.at[pl.ds(base, STRIPE)],
                            out_hbm.at[c, pl.ds(base, STRIPE)])

        @pl.when(s == NS - 1)
        def _():
            pltpu.sync_copy(agg_sh.at[pl.ds(base, LAST)],
                            out_hbm.at[c, pl.ds(base, LAST)])

    return k(x, sd, wp, zrows)


def _tc_layer(partials, x, wrel_t, wroot_t, b2d, relu):
    N, D = x.shape
    BN = 1000
    assert N % BN == 0

    def body(p_ref, x_ref, wr_ref, wt_ref, b_ref, o_ref):
        agg = p_ref[0] + p_ref[1]
        acc = jnp.dot(agg, wr_ref[...], preferred_element_type=jnp.float32)
        acc = acc + jnp.dot(x_ref[...], wt_ref[...],
                            preferred_element_type=jnp.float32)
        acc = acc + b_ref[...]
        if relu:
            acc = jnp.maximum(acc, 0.0)
        o_ref[...] = acc

    return pl.pallas_call(
        body,
        grid=(N // BN,),
        in_specs=[
            pl.BlockSpec((NC, BN, D), lambda i: (0, i, 0)),
            pl.BlockSpec((BN, D), lambda i: (i, 0)),
            pl.BlockSpec((D, D), lambda i: (0, 0)),
            pl.BlockSpec((D, D), lambda i: (0, 0)),
            pl.BlockSpec((1, D), lambda i: (0, 0)),
        ],
        out_specs=pl.BlockSpec((BN, D), lambda i: (i, 0)),
        out_shape=jax.ShapeDtypeStruct((N, D), jnp.float32),
    )(partials, x, wrel_t, wroot_t, b2d)


def kernel(x, edge_index, edge_weight, batch,
           W1_rel, b1_rel, W1_root, W2_rel, b2_rel, W2_root,
           W3_rel, b3_rel, W3_root):
    del batch
    N, D = x.shape
    E = edge_index.shape[1]

    K = -(-E // (NW * CHUNK))
    K = -(-K // NBUF) * NBUF
    e_pad = K * CHUNK * NW
    pad = e_pad - E
    src = jnp.concatenate(
        [edge_index[0], jnp.zeros((pad,), jnp.int32)]).reshape(NW, K, 1, CHUNK)
    dst = jnp.concatenate(
        [edge_index[1], jnp.zeros((pad,), jnp.int32)]).reshape(NW, K, 1, CHUNK)
    sd = jnp.concatenate([src, dst], axis=2)
    w = jnp.concatenate(
        [edge_weight, jnp.zeros((pad,), jnp.float32)]).reshape(NW, K, CHUNK)
    zrows = jnp.zeros((640, D), jnp.float32)

    h = x
    layers = [
        (W1_rel, b1_rel, W1_root, True),
        (W2_rel, b2_rel, W2_root, True),
        (W3_rel, b3_rel, W3_root, False),
    ]
    for W_rel, b_rel, W_root, relu in layers:
        partials = _sc_agg(h, sd, w, zrows)
        h = _tc_layer(partials, h, W_rel.T, W_root.T, b_rel[None, :], relu)
    return h
